# baseline (speedup 1.0000x reference)
import functools

import jax
import jax.numpy as jnp
from jax import lax
from jax.experimental import pallas as pl
from jax.experimental.pallas import tpu as pltpu
from jax.experimental.pallas import tpu_sc as plsc

_B, _N, _C, _M, _S = 4, 16384, 128, 128, 512
_D = 3 + _C
_EXTRA = 1.0
_L = 16
_NW = 32
_BPW = (_B * _M) // _NW
_WPB = _M // _BPW
_NVEC = _N // _L
_CHUNK = 8
_NP1 = _N + 1
_DP = 144

_GDN = lax.GatherDimensionNumbers(
    offset_dims=(), collapsed_slice_dims=(0,), start_index_map=(0,))


def _vpermute(x, idx):
    return lax.gather(x, idx[:, None], _GDN, slice_sizes=(1,),
                      mode=lax.GatherScatterMode.PROMISE_IN_BOUNDS)


def _sc_body(xs, ys, zs, boxp, data, out, flags_out,
             xs_v, ys_v, zs_v, boxp_v, idx_buf, pidx_buf, rows_v, flags_v,
             gsem, osem):
    cid = lax.axis_index("c")
    sid = lax.axis_index("s")
    wid = sid * 2 + cid
    b = wid // _WPB
    mg = (wid % _WPB) * _BPW

    pltpu.sync_copy(xs.at[b], xs_v)
    pltpu.sync_copy(ys.at[b], ys_v)
    pltpu.sync_copy(zs.at[b], zs_v)

    lane = lax.iota(jnp.int32, _L)

    def box_body(k, flag_vec):
        m = mg + k
        pltpu.sync_copy(boxp.at[b, m], boxp_v)
        cx = boxp_v[0]
        cy = boxp_v[1]
        cz = boxp_v[2]
        hx = boxp_v[3]
        hy = boxp_v[4]
        hz = boxp_v[5]
        ca = boxp_v[6]
        sa = boxp_v[7]

        s_cap = jnp.full((_L,), _S, jnp.int32)
        ones_v = jnp.full((_L,), 1, jnp.int32)
        zeros_v = jnp.full((_L,), 0, jnp.int32)

        def scan_cond(carry):
            i, off_v = carry
            return (i < _NVEC // _CHUNK) & jnp.all(off_v < s_cap)

        def scan_body(carry):
            i, off_v = carry
            base = i * (_CHUNK * _L)
            for t in range(_CHUNK):
                xv = xs_v[pl.ds(base + t * _L, _L)]
                yv = ys_v[pl.ds(base + t * _L, _L)]
                zv = zs_v[pl.ds(base + t * _L, _L)]
                px = xv - cx
                py = yv - cy
                pz = zv - cz
                lx = px * ca + py * sa
                ly = py * ca - px * sa
                mask = ((jnp.abs(lx) < hx) & (jnp.abs(ly) < hy)
                        & (jnp.abs(pz) <= hz))
                mi = jnp.where(mask, ones_v, zeros_v)
                pos = plsc.cumsum(mi) + off_v - ones_v
                ok = mask & (pos < s_cap)
                ptid = lane + jnp.full((_L,), t * _L, jnp.int32) \
                    + jnp.full((_L,), base, jnp.int32)
                plsc.store_scatter(idx_buf, [pos], ptid, mask=ok)
                off_v = off_v + plsc.all_reduce_population_count(mask)
            return i + 1, off_v

        _, cnt_all_v = lax.while_loop(
            scan_cond, scan_body, (jnp.int32(0), jnp.zeros((_L,), jnp.int32)))

        cnt_v = jnp.minimum(cnt_all_v, s_cap)
        cntc_v = jnp.maximum(cnt_v, ones_v)
        nonempty = cnt_v > zeros_v
        base_v = jnp.full((_L,), b * _NP1, jnp.int32)
        zrow_v = jnp.full((_L,), b * _NP1 + _N, jnp.int32)
        for j in range(_S // _L):
            sv = lane + jnp.full((_L,), j * _L, jnp.int32)
            rv = jnp.remainder(sv, cntc_v)
            g = plsc.load_gather(idx_buf, [rv])
            g = jnp.where(nonempty, g + base_v, zrow_v)
            pidx_buf[j // 8, pl.ds((j % 8) * _L, _L)] = g

        @pl.when(k > 0)
        def _():
            pltpu.make_async_copy(rows_v, out.at[b, m - 1], osem).wait()

        descs = [
            pltpu.async_copy(data.at[pidx_buf.at[j]],
                             rows_v.at[pl.ds(j * 128, 128)], gsem)
            for j in range(_S // 128)
        ]
        for d in descs:
            d.wait()

        pltpu.async_copy(rows_v, out.at[b, m], osem)

        flag = jnp.where(cnt_all_v == zeros_v, ones_v, zeros_v)
        k_v = jnp.full((_L,), k, jnp.int32)
        return jnp.where(lane == k_v, flag, flag_vec)

    flags = lax.fori_loop(0, _BPW, box_body, jnp.zeros((_L,), jnp.int32))
    pltpu.make_async_copy(rows_v, out.at[b, mg + _BPW - 1], osem).wait()
    flags_v[...] = flags
    pltpu.sync_copy(flags_v, flags_out.at[b, pl.ds(mg, _L)])


@functools.partial(
    pl.kernel,
    mesh=plsc.VectorSubcoreMesh(core_axis_name="c", subcore_axis_name="s"),
    compiler_params=pltpu.CompilerParams(needs_layout_passes=False,
                                         use_tc_tiling_on_sc=False),
    out_type=[
        jax.ShapeDtypeStruct((_B, _M, _S, _DP), jnp.float32),
        jax.ShapeDtypeStruct((_B, _M), jnp.int32),
    ],
    scratch_types=[
        pltpu.VMEM((_N,), jnp.float32),
        pltpu.VMEM((_N,), jnp.float32),
        pltpu.VMEM((_N,), jnp.float32),
        pltpu.VMEM((8, _L), jnp.float32),
        pltpu.VMEM((_S + _L,), jnp.int32),
        pltpu.VMEM((_S // 128, 128), jnp.int32),
        pltpu.VMEM((_S, _DP), jnp.float32),
        pltpu.VMEM((_L,), jnp.int32),
        pltpu.SemaphoreType.DMA,
        pltpu.SemaphoreType.DMA,
    ],
)
def _sc_pool(*args):
    _sc_body(*args)


def kernel(points, point_features, boxes3d):
    points = points.astype(jnp.float32)
    point_features = point_features.astype(jnp.float32)
    boxes3d = boxes3d.astype(jnp.float32)

    xs = points[..., 0]
    ys = points[..., 1]
    zs = points[..., 2]
    half = boxes3d[..., 3:6] * 0.5 + _EXTRA
    rz = boxes3d[..., 6]
    boxp = jnp.stack(
        [boxes3d[..., 0], boxes3d[..., 1], boxes3d[..., 2],
         half[..., 0], half[..., 1], half[..., 2],
         jnp.cos(rz), jnp.sin(rz)], axis=-1)
    boxp = (boxp[..., None] * jnp.ones((_L,), jnp.float32))

    data = jnp.concatenate([points, point_features], axis=-1)
    data = jnp.pad(data, ((0, 0), (0, 1), (0, _DP - _D)))
    data = data.reshape(_B * _NP1, _DP)

    pooled, flags = _sc_pool(xs, ys, zs, boxp, data)
    return pooled, flags


# speedup vs baseline: 1.2229x; 1.2229x over previous
import functools

import jax
import jax.numpy as jnp
from jax import lax
from jax.experimental import pallas as pl
from jax.experimental.pallas import tpu as pltpu
from jax.experimental.pallas import tpu_sc as plsc

_B, _N, _C, _M, _S = 4, 16384, 128, 128, 512
_D = 3 + _C
_EXTRA = 1.0
_L = 16
_NW = 32
_BPW = (_B * _M) // _NW
_WPB = _M // _BPW
_CHUNK = 8
_NIT = _N // (_CHUNK * _L)
_H = _S // 2
_HW = _H * _D


def _sc_body(xs, ys, zs, boxp, feat, out, flags_out,
             xs_v, ys_v, zs_v, boxp_v, idx_buf, pidx_buf, rows_f, flat_v,
             flags_v, gsem, osem):
    cid = lax.axis_index("c")
    sid = lax.axis_index("s")
    wid = sid * 2 + cid
    b = wid // _WPB
    mg = (wid % _WPB) * _BPW

    pltpu.sync_copy(xs.at[b], xs_v)
    pltpu.sync_copy(ys.at[b], ys_v)
    pltpu.sync_copy(zs.at[b], zs_v)

    lane = lax.iota(jnp.int32, _L)
    ones_v = jnp.full((_L,), 1, jnp.int32)
    zeros_v = jnp.full((_L,), 0, jnp.int32)
    s_cap = jnp.full((_L,), _S, jnp.int32)
    d_v = jnp.full((_L,), _D, jnp.int32)
    bn_v = jnp.full((_L,), b * _N, jnp.int32)
    seven_v = jnp.full((_L,), 7, jnp.int32)
    m127_v = jnp.full((_L,), 127, jnp.int32)
    fzero_v = jnp.zeros((_L,), jnp.float32)

    def box_body(k, flag_vec):
        m = mg + k
        pltpu.sync_copy(boxp.at[b, m], boxp_v)
        cx = boxp_v[0]
        cy = boxp_v[1]
        cz = boxp_v[2]
        hx = boxp_v[3]
        hy = boxp_v[4]
        hz = boxp_v[5]
        ca = boxp_v[6]
        sa = boxp_v[7]

        def scan_cond(carry):
            i, off_v = carry
            return (i < _NIT) & jnp.all(off_v < s_cap)

        def scan_body(carry):
            i, off_v = carry
            base = i * (_CHUNK * _L)
            for t in range(_CHUNK):
                xv = xs_v[i, pl.ds(t * _L, _L)]
                yv = ys_v[i, pl.ds(t * _L, _L)]
                zv = zs_v[i, pl.ds(t * _L, _L)]
                px = xv - cx
                py = yv - cy
                pz = zv - cz
                lx = px * ca + py * sa
                ly = py * ca - px * sa
                mask = ((jnp.abs(lx) < hx) & (jnp.abs(ly) < hy)
                        & (jnp.abs(pz) <= hz))
                mi = jnp.where(mask, ones_v, zeros_v)
                pos = plsc.cumsum(mi) + off_v - ones_v
                ok = mask & (pos < s_cap)
                ptid = lane + jnp.full((_L,), t * _L, jnp.int32) \
                    + jnp.full((_L,), base, jnp.int32)
                plsc.store_scatter(idx_buf, [pos], ptid, mask=ok)
                off_v = off_v + plsc.all_reduce_population_count(mask)
            return i + 1, off_v

        _, cnt_all_v = lax.while_loop(
            scan_cond, scan_body, (jnp.int32(0), jnp.zeros((_L,), jnp.int32)))

        cnt_v = jnp.minimum(cnt_all_v, s_cap)
        cntc_v = jnp.maximum(cnt_v, ones_v)
        nonempty = cnt_v > zeros_v
        ne = jnp.all(nonempty)
        for j in range(_S // _L):
            sv = lane + jnp.full((_L,), j * _L, jnp.int32)
            rv = jnp.remainder(sv, cntc_v)
            g = plsc.load_gather(idx_buf, [rv])
            g = jnp.where(nonempty, g + bn_v, bn_v)
            pidx_buf[j // 8, pl.ds((j % 8) * _L, _L)] = g

        for h in range(2):
            for j in range(2):
                pltpu.async_copy(
                    feat.at[pidx_buf.at[2 * h + j]],
                    rows_f.at[pl.ds(j * 128, 128)], gsem)
            for j in range(2):
                pltpu.make_async_copy(
                    feat.at[pidx_buf.at[2 * h + j]],
                    rows_f.at[pl.ds(j * 128, 128)], gsem).wait()

            if h == 0:
                @pl.when(k > 0)
                def _():
                    pltpu.make_async_copy(
                        flat_v, out.at[b, m - 1, pl.ds(_HW, _HW)],
                        osem).wait()
            else:
                pltpu.make_async_copy(
                    flat_v, out.at[b, m, pl.ds(0, _HW)], osem).wait()

            @pl.when(ne)
            def _():
                def frow(s, acc):
                    base = s * _D + 3
                    for j2 in range(8):
                        v = rows_f[s, pl.ds(j2 * _L, _L)]
                        idxv = lane + jnp.full((_L,), base + j2 * _L,
                                               jnp.int32)
                        plsc.store_scatter(flat_v, [idxv], v)
                    return acc

                lax.fori_loop(0, _H, frow, jnp.int32(0))
                for t in range(_H // _L):
                    jj = 2 * h + t // 8
                    pv = pidx_buf[jj, pl.ds((t % 8) * _L, _L)] - bn_v
                    rr = jax.lax.shift_right_logical(pv, seven_v)
                    cc = pv & m127_v
                    xg = plsc.load_gather(xs_v, [rr, cc])
                    yg = plsc.load_gather(ys_v, [rr, cc])
                    zg = plsc.load_gather(zs_v, [rr, cc])
                    sv = lane + jnp.full((_L,), t * _L, jnp.int32)
                    p0 = sv * d_v
                    plsc.store_scatter(flat_v, [p0], xg)
                    plsc.store_scatter(flat_v, [p0 + ones_v], yg)
                    plsc.store_scatter(flat_v, [p0 + ones_v + ones_v], zg)

            @pl.when(jnp.logical_not(ne))
            def _():
                def zrow(q, acc):
                    flat_v[pl.ds(q * _L, _L)] = fzero_v
                    return acc

                lax.fori_loop(0, _HW // _L, zrow, jnp.int32(0))

            pltpu.async_copy(flat_v, out.at[b, m, pl.ds(h * _HW, _HW)], osem)

        flag = jnp.where(nonempty, zeros_v, ones_v)
        k_v = jnp.full((_L,), k, jnp.int32)
        return jnp.where(lane == k_v, flag, flag_vec)

    flags = lax.fori_loop(0, _BPW, box_body, jnp.zeros((_L,), jnp.int32))
    pltpu.make_async_copy(
        flat_v, out.at[b, mg + _BPW - 1, pl.ds(_HW, _HW)], osem).wait()
    flags_v[...] = flags
    pltpu.sync_copy(flags_v, flags_out.at[b, pl.ds(mg, _L)])


@functools.partial(
    pl.kernel,
    mesh=plsc.VectorSubcoreMesh(core_axis_name="c", subcore_axis_name="s"),
    compiler_params=pltpu.CompilerParams(needs_layout_passes=False,
                                         use_tc_tiling_on_sc=False),
    out_type=[
        jax.ShapeDtypeStruct((_B, _M, _S * _D), jnp.float32),
        jax.ShapeDtypeStruct((_B, _M), jnp.int32),
    ],
    scratch_types=[
        pltpu.VMEM((_N // 128, 128), jnp.float32),
        pltpu.VMEM((_N // 128, 128), jnp.float32),
        pltpu.VMEM((_N // 128, 128), jnp.float32),
        pltpu.VMEM((8, _L), jnp.float32),
        pltpu.VMEM((_S,), jnp.int32),
        pltpu.VMEM((_S // 128, 128), jnp.int32),
        pltpu.VMEM((_H, 128), jnp.float32),
        pltpu.VMEM((_HW,), jnp.float32),
        pltpu.VMEM((_L,), jnp.int32),
        pltpu.SemaphoreType.DMA,
        pltpu.SemaphoreType.DMA,
    ],
)
def _sc_pool(*args):
    _sc_body(*args)


def kernel(points, point_features, boxes3d):
    points = points.astype(jnp.float32)
    point_features = point_features.astype(jnp.float32)
    boxes3d = boxes3d.astype(jnp.float32)

    xs = points[..., 0].reshape(_B, _N // 128, 128)
    ys = points[..., 1].reshape(_B, _N // 128, 128)
    zs = points[..., 2].reshape(_B, _N // 128, 128)
    half = boxes3d[..., 3:6] * 0.5 + _EXTRA
    rz = boxes3d[..., 6]
    boxp = jnp.stack(
        [boxes3d[..., 0], boxes3d[..., 1], boxes3d[..., 2],
         half[..., 0], half[..., 1], half[..., 2],
         jnp.cos(rz), jnp.sin(rz)], axis=-1)
    boxp = (boxp[..., None] * jnp.ones((_L,), jnp.float32))

    feat = point_features.reshape(_B * _N, _C)

    pooled, flags = _sc_pool(xs, ys, zs, boxp, feat)
    return pooled.reshape(_B, _M, _S, _D), flags


# speedup vs baseline: 1.2312x; 1.0068x over previous
import functools

import jax
import jax.numpy as jnp
from jax import lax
from jax.experimental import pallas as pl
from jax.experimental.pallas import tpu as pltpu
from jax.experimental.pallas import tpu_sc as plsc

_B, _N, _C, _M, _S = 4, 16384, 128, 128, 512
_D = 3 + _C
_EXTRA = 1.0
_L = 16
_NW = 32
_BPW = (_B * _M) // _NW
_WPB = _M // _BPW
_CHUNK = 8
_NIT = _N // (_CHUNK * _L)
_H = _S // 2
_HW = _H * _D


def _sc_body(xs, ys, zs, boxp, feat, out, flags_out,
             xs_v, ys_v, zs_v, boxp_v, idx_buf, pidx_buf, rows_f, flat_v,
             flags_v, gsem, osem):
    cid = lax.axis_index("c")
    sid = lax.axis_index("s")
    wid = sid * 2 + cid
    b = wid // _WPB
    mg = (wid % _WPB) * _BPW

    pltpu.sync_copy(xs.at[b], xs_v)
    pltpu.sync_copy(ys.at[b], ys_v)
    pltpu.sync_copy(zs.at[b], zs_v)

    lane = lax.iota(jnp.int32, _L)
    ones_v = jnp.full((_L,), 1, jnp.int32)
    zeros_v = jnp.full((_L,), 0, jnp.int32)
    s_cap = jnp.full((_L,), _S, jnp.int32)
    d_v = jnp.full((_L,), _D, jnp.int32)
    bn_v = jnp.full((_L,), b * _N, jnp.int32)
    seven_v = jnp.full((_L,), 7, jnp.int32)
    m127_v = jnp.full((_L,), 127, jnp.int32)
    fzero_v = jnp.zeros((_L,), jnp.float32)

    def box_body(k, flag_vec):
        m = mg + k
        pltpu.sync_copy(boxp.at[b, m], boxp_v)
        cx = boxp_v[0]
        cy = boxp_v[1]
        cz = boxp_v[2]
        hx = boxp_v[3]
        hy = boxp_v[4]
        hz = boxp_v[5]
        ca = boxp_v[6]
        sa = boxp_v[7]

        def scan_cond(carry):
            i, off_v = carry
            return (i < _NIT) & jnp.all(off_v < s_cap)

        def scan_body(carry):
            i, off_v = carry
            base = i * (_CHUNK * _L)
            for t in range(_CHUNK):
                xv = xs_v[i, pl.ds(t * _L, _L)]
                yv = ys_v[i, pl.ds(t * _L, _L)]
                zv = zs_v[i, pl.ds(t * _L, _L)]
                px = xv - cx
                py = yv - cy
                pz = zv - cz
                lx = px * ca + py * sa
                ly = py * ca - px * sa
                mask = ((jnp.abs(lx) < hx) & (jnp.abs(ly) < hy)
                        & (jnp.abs(pz) <= hz))
                mi = jnp.where(mask, ones_v, zeros_v)
                pos = plsc.cumsum(mi) + off_v - ones_v
                ok = mask & (pos < s_cap)
                ptid = lane + jnp.full((_L,), t * _L, jnp.int32) \
                    + jnp.full((_L,), base, jnp.int32)
                plsc.store_scatter(idx_buf, [pos], ptid, mask=ok)
                off_v = off_v + plsc.all_reduce_population_count(mask)
            return i + 1, off_v

        _, cnt_all_v = lax.while_loop(
            scan_cond, scan_body, (jnp.int32(0), jnp.zeros((_L,), jnp.int32)))

        cnt_v = jnp.minimum(cnt_all_v, s_cap)
        cntc_v = jnp.maximum(cnt_v, ones_v)
        nonempty = cnt_v > zeros_v
        ne = jnp.all(nonempty)
        for j in range(_S // _L):
            sv = lane + jnp.full((_L,), j * _L, jnp.int32)
            rv = jnp.remainder(sv, cntc_v)
            g = plsc.load_gather(idx_buf, [rv])
            g = jnp.where(nonempty, g + bn_v, bn_v)
            pidx_buf[j // 8, pl.ds((j % 8) * _L, _L)] = g

        for h in range(2):
            for j in range(2):
                pltpu.async_copy(
                    feat.at[pidx_buf.at[2 * h + j]],
                    rows_f.at[pl.ds(j * 128, 128)], gsem)
            for j in range(2):
                pltpu.make_async_copy(
                    feat.at[pidx_buf.at[2 * h + j]],
                    rows_f.at[pl.ds(j * 128, 128)], gsem).wait()

            if h == 0:
                @pl.when(k > 0)
                def _():
                    pltpu.make_async_copy(
                        flat_v, out.at[b, m - 1, pl.ds(_HW, _HW)],
                        osem).wait()
            else:
                pltpu.make_async_copy(
                    flat_v, out.at[b, m, pl.ds(0, _HW)], osem).wait()

            @pl.when(ne)
            def _():
                dstep = jnp.full((_L,), 4 * _D, jnp.int32)

                def frow(s4, vb):
                    for r in range(4):
                        s = s4 * 4 + r
                        for j2 in range(8):
                            v = rows_f[s, pl.ds(j2 * _L, _L)]
                            idxv = vb + jnp.full((_L,), r * _D + j2 * _L,
                                                 jnp.int32)
                            plsc.store_scatter(flat_v, [idxv], v)
                    return vb + dstep

                lax.fori_loop(0, _H // 4, frow,
                              lane + jnp.full((_L,), 3, jnp.int32))
                for t in range(_H // _L):
                    jj = 2 * h + t // 8
                    pv = pidx_buf[jj, pl.ds((t % 8) * _L, _L)] - bn_v
                    rr = jax.lax.shift_right_logical(pv, seven_v)
                    cc = pv & m127_v
                    xg = plsc.load_gather(xs_v, [rr, cc])
                    yg = plsc.load_gather(ys_v, [rr, cc])
                    zg = plsc.load_gather(zs_v, [rr, cc])
                    sv = lane + jnp.full((_L,), t * _L, jnp.int32)
                    p0 = sv * d_v
                    plsc.store_scatter(flat_v, [p0], xg)
                    plsc.store_scatter(flat_v, [p0 + ones_v], yg)
                    plsc.store_scatter(flat_v, [p0 + ones_v + ones_v], zg)

            @pl.when(jnp.logical_not(ne))
            def _():
                def zrow(q, acc):
                    flat_v[pl.ds(q * _L, _L)] = fzero_v
                    return acc

                lax.fori_loop(0, _HW // _L, zrow, jnp.int32(0))

            pltpu.async_copy(flat_v, out.at[b, m, pl.ds(h * _HW, _HW)], osem)

        flag = jnp.where(nonempty, zeros_v, ones_v)
        k_v = jnp.full((_L,), k, jnp.int32)
        return jnp.where(lane == k_v, flag, flag_vec)

    flags = lax.fori_loop(0, _BPW, box_body, jnp.zeros((_L,), jnp.int32))
    pltpu.make_async_copy(
        flat_v, out.at[b, mg + _BPW - 1, pl.ds(_HW, _HW)], osem).wait()
    flags_v[...] = flags
    pltpu.sync_copy(flags_v, flags_out.at[b, pl.ds(mg, _L)])


@functools.partial(
    pl.kernel,
    mesh=plsc.VectorSubcoreMesh(core_axis_name="c", subcore_axis_name="s"),
    compiler_params=pltpu.CompilerParams(needs_layout_passes=False,
                                         use_tc_tiling_on_sc=False),
    out_type=[
        jax.ShapeDtypeStruct((_B, _M, _S * _D), jnp.float32),
        jax.ShapeDtypeStruct((_B, _M), jnp.int32),
    ],
    scratch_types=[
        pltpu.VMEM((_N // 128, 128), jnp.float32),
        pltpu.VMEM((_N // 128, 128), jnp.float32),
        pltpu.VMEM((_N // 128, 128), jnp.float32),
        pltpu.VMEM((8, _L), jnp.float32),
        pltpu.VMEM((_S,), jnp.int32),
        pltpu.VMEM((_S // 128, 128), jnp.int32),
        pltpu.VMEM((_H, 128), jnp.float32),
        pltpu.VMEM((_HW,), jnp.float32),
        pltpu.VMEM((_L,), jnp.int32),
        pltpu.SemaphoreType.DMA,
        pltpu.SemaphoreType.DMA,
    ],
)
def _sc_pool(*args):
    _sc_body(*args)


def kernel(points, point_features, boxes3d):
    points = points.astype(jnp.float32)
    point_features = point_features.astype(jnp.float32)
    boxes3d = boxes3d.astype(jnp.float32)

    xs = points[..., 0].reshape(_B, _N // 128, 128)
    ys = points[..., 1].reshape(_B, _N // 128, 128)
    zs = points[..., 2].reshape(_B, _N // 128, 128)
    half = boxes3d[..., 3:6] * 0.5 + _EXTRA
    rz = boxes3d[..., 6]
    boxp = jnp.stack(
        [boxes3d[..., 0], boxes3d[..., 1], boxes3d[..., 2],
         half[..., 0], half[..., 1], half[..., 2],
         jnp.cos(rz), jnp.sin(rz)], axis=-1)
    boxp = (boxp[..., None] * jnp.ones((_L,), jnp.float32))

    feat = point_features.reshape(_B * _N, _C)

    pooled, flags = _sc_pool(xs, ys, zs, boxp, feat)
    return pooled.reshape(_B, _M, _S, _D), flags


# speedup vs baseline: 1.2454x; 1.0116x over previous
import functools

import jax
import jax.numpy as jnp
from jax import lax
from jax.experimental import pallas as pl
from jax.experimental.pallas import tpu as pltpu
from jax.experimental.pallas import tpu_sc as plsc

_B, _N, _C, _M, _S = 4, 16384, 128, 128, 512
_D = 3 + _C
_EXTRA = 1.0
_L = 16
_NW = 32
_BPW = (_B * _M) // _NW
_WPB = _M // _BPW
_CHUNK = 8
_NIT = _N // (_CHUNK * _L)
_Q = _S // 4


def _sc_body(xs, ys, zs, boxp, feat, out, flags_out,
             xs_v, ys_v, zs_v, boxp_v, idx_buf, pidx_buf, rows_f, flat_v,
             flags_v, gsem, osem):
    cid = lax.axis_index("c")
    sid = lax.axis_index("s")
    wid = sid * 2 + cid
    b = wid // _WPB
    mg = (wid % _WPB) * _BPW

    pltpu.sync_copy(xs.at[b], xs_v)
    pltpu.sync_copy(ys.at[b], ys_v)
    pltpu.sync_copy(zs.at[b], zs_v)

    lane = lax.iota(jnp.int32, _L)
    ones_v = jnp.full((_L,), 1, jnp.int32)
    zeros_v = jnp.full((_L,), 0, jnp.int32)
    s_cap = jnp.full((_L,), _S, jnp.int32)
    d_v = jnp.full((_L,), _D, jnp.int32)
    bn_v = jnp.full((_L,), b * _N, jnp.int32)
    seven_v = jnp.full((_L,), 7, jnp.int32)
    m127_v = jnp.full((_L,), 127, jnp.int32)
    fzero_v = jnp.zeros((_L,), jnp.float32)

    def box_body(k, flag_vec):
        m = mg + k
        pltpu.sync_copy(boxp.at[b, m], boxp_v)
        cx = boxp_v[0]
        cy = boxp_v[1]
        cz = boxp_v[2]
        hx = boxp_v[3]
        hy = boxp_v[4]
        hz = boxp_v[5]
        ca = boxp_v[6]
        sa = boxp_v[7]

        def scan_cond(carry):
            i, off_v = carry
            return (i < _NIT) & jnp.all(off_v < s_cap)

        def scan_body(carry):
            i, off_v = carry
            base = i * (_CHUNK * _L)
            for t in range(_CHUNK):
                xv = xs_v[i, pl.ds(t * _L, _L)]
                yv = ys_v[i, pl.ds(t * _L, _L)]
                zv = zs_v[i, pl.ds(t * _L, _L)]
                px = xv - cx
                py = yv - cy
                pz = zv - cz
                lx = px * ca + py * sa
                ly = py * ca - px * sa
                mask = ((jnp.abs(lx) < hx) & (jnp.abs(ly) < hy)
                        & (jnp.abs(pz) <= hz))
                mi = jnp.where(mask, ones_v, zeros_v)
                pos = plsc.cumsum(mi) + off_v - ones_v
                ok = mask & (pos < s_cap)
                ptid = lane + jnp.full((_L,), t * _L, jnp.int32) \
                    + jnp.full((_L,), base, jnp.int32)
                plsc.store_scatter(idx_buf, [pos], ptid, mask=ok)
                off_v = off_v + plsc.all_reduce_population_count(mask)
            return i + 1, off_v

        _, cnt_all_v = lax.while_loop(
            scan_cond, scan_body, (jnp.int32(0), jnp.zeros((_L,), jnp.int32)))

        cnt_v = jnp.minimum(cnt_all_v, s_cap)
        cntc_v = jnp.maximum(cnt_v, ones_v)
        nonempty = cnt_v > zeros_v
        ne = jnp.all(nonempty)
        for j in range(_S // _L):
            sv = lane + jnp.full((_L,), j * _L, jnp.int32)
            rv = jnp.remainder(sv, cntc_v)
            g = plsc.load_gather(idx_buf, [rv])
            g = jnp.where(nonempty, g + bn_v, bn_v)
            pidx_buf[j // 8, pl.ds((j % 8) * _L, _L)] = g

        def gdesc(q):
            return pltpu.make_async_copy(
                feat.at[pidx_buf.at[q]], rows_f.at[q % 2], gsem)

        def odesc(mm, q, qb):
            return pltpu.make_async_copy(
                flat_v.at[qb], out.at[b, mm, pl.ds(q * _Q, _Q)], osem)

        pltpu.async_copy(feat.at[pidx_buf.at[0]], rows_f.at[0], gsem)
        for q in range(4):
            qb = q % 2
            gdesc(q).wait()
            if q < 3:
                pltpu.async_copy(feat.at[pidx_buf.at[q + 1]],
                                 rows_f.at[(q + 1) % 2], gsem)
            if q >= 2:
                odesc(m, q - 2, qb).wait()
            else:
                @pl.when(k > 0)
                def _():
                    odesc(m - 1, q + 2, qb).wait()

            @pl.when(ne)
            def _():
                def frow(s4, rbase):
                    for r in range(4):
                        s = s4 * 4 + r
                        for j2 in range(8):
                            v = rows_f[qb, s, pl.ds(j2 * _L, _L)]
                            cidx = lane + jnp.full((_L,), 3 + j2 * _L,
                                                   jnp.int32)
                            ridx = rbase + jnp.full((_L,), r, jnp.int32)
                            plsc.store_scatter(flat_v.at[qb], [ridx, cidx], v)
                    return rbase + jnp.full((_L,), 4, jnp.int32)

                lax.fori_loop(0, _Q // 4, frow, jnp.zeros((_L,), jnp.int32))
                for t in range(_Q // _L):
                    pv = pidx_buf[q, pl.ds(t * _L, _L)] - bn_v
                    rr = jax.lax.shift_right_logical(pv, seven_v)
                    cc = pv & m127_v
                    xg = plsc.load_gather(xs_v, [rr, cc])
                    yg = plsc.load_gather(ys_v, [rr, cc])
                    zg = plsc.load_gather(zs_v, [rr, cc])
                    sv = lane + jnp.full((_L,), t * _L, jnp.int32)
                    plsc.store_scatter(flat_v.at[qb], [sv, zeros_v], xg)
                    plsc.store_scatter(flat_v.at[qb], [sv, ones_v], yg)
                    plsc.store_scatter(flat_v.at[qb], [sv, ones_v + ones_v],
                                       zg)

            @pl.when(jnp.logical_not(ne))
            def _():
                tail_ok = lane < jnp.full((_L,), 3, jnp.int32)

                def zrow(s, acc):
                    for j2 in range(8):
                        flat_v[qb, s, pl.ds(j2 * _L, _L)] = fzero_v
                    tidx = lane + jnp.full((_L,), 128, jnp.int32)
                    plsc.store_scatter(flat_v.at[qb],
                                       [jnp.full((_L,), s, jnp.int32), tidx],
                                       fzero_v, mask=tail_ok)
                    return acc

                lax.fori_loop(0, _Q, zrow, jnp.int32(0))

            pltpu.async_copy(flat_v.at[qb], out.at[b, m, pl.ds(q * _Q, _Q)],
                             osem)

        flag = jnp.where(nonempty, zeros_v, ones_v)
        k_v = jnp.full((_L,), k, jnp.int32)
        return jnp.where(lane == k_v, flag, flag_vec)

    flags = lax.fori_loop(0, _BPW, box_body, jnp.zeros((_L,), jnp.int32))
    m_last = mg + _BPW - 1
    for q in (2, 3):
        pltpu.make_async_copy(
            flat_v.at[q % 2], out.at[b, m_last, pl.ds(q * _Q, _Q)],
            osem).wait()
    flags_v[...] = flags
    pltpu.sync_copy(flags_v, flags_out.at[b, pl.ds(mg, _L)])


@functools.partial(
    pl.kernel,
    mesh=plsc.VectorSubcoreMesh(core_axis_name="c", subcore_axis_name="s"),
    compiler_params=pltpu.CompilerParams(needs_layout_passes=False,
                                         use_tc_tiling_on_sc=False),
    out_type=[
        jax.ShapeDtypeStruct((_B, _M, _S, _D), jnp.float32),
        jax.ShapeDtypeStruct((_B, _M), jnp.int32),
    ],
    scratch_types=[
        pltpu.VMEM((_N // 128, 128), jnp.float32),
        pltpu.VMEM((_N // 128, 128), jnp.float32),
        pltpu.VMEM((_N // 128, 128), jnp.float32),
        pltpu.VMEM((8, _L), jnp.float32),
        pltpu.VMEM((_S,), jnp.int32),
        pltpu.VMEM((_S // 128, 128), jnp.int32),
        pltpu.VMEM((2, _Q, 128), jnp.float32),
        pltpu.VMEM((2, _Q, _D), jnp.float32),
        pltpu.VMEM((_L,), jnp.int32),
        pltpu.SemaphoreType.DMA,
        pltpu.SemaphoreType.DMA,
    ],
)
def _sc_pool(*args):
    _sc_body(*args)


def kernel(points, point_features, boxes3d):
    points = points.astype(jnp.float32)
    point_features = point_features.astype(jnp.float32)
    boxes3d = boxes3d.astype(jnp.float32)

    xs = points[..., 0].reshape(_B, _N // 128, 128)
    ys = points[..., 1].reshape(_B, _N // 128, 128)
    zs = points[..., 2].reshape(_B, _N // 128, 128)
    half = boxes3d[..., 3:6] * 0.5 + _EXTRA
    rz = boxes3d[..., 6]
    boxp = jnp.stack(
        [boxes3d[..., 0], boxes3d[..., 1], boxes3d[..., 2],
         half[..., 0], half[..., 1], half[..., 2],
         jnp.cos(rz), jnp.sin(rz)], axis=-1)
    boxp = (boxp[..., None] * jnp.ones((_L,), jnp.float32))

    feat = point_features.reshape(_B * _N, _C)

    pooled, flags = _sc_pool(xs, ys, zs, boxp, feat)
    return pooled, flags


# speedup vs baseline: 1.6440x; 1.3200x over previous
import functools

import jax
import jax.numpy as jnp
from jax import lax
from jax.experimental import pallas as pl
from jax.experimental.pallas import tpu as pltpu
from jax.experimental.pallas import tpu_sc as plsc

_B, _N, _C, _M, _S = 4, 16384, 128, 128, 512
_D = 3 + _C
_EXTRA = 1.0
_L = 16
_NW = 32
_BPW = (_B * _M) // _NW
_WPB = _M // _BPW
_CHUNK = 8
_NIT = _N // (_CHUNK * _L)
_Q = _S // 4


def _sc_body(xs, ys, zs, boxp, feat, out, flags_out,
             xs_v, ys_v, zs_v, boxp_v, idx_buf, pidx_buf, rows_f, flat_v,
             flags_v, gsem, osem):
    cid = lax.axis_index("c")
    sid = lax.axis_index("s")
    wid = sid * 2 + cid
    b = wid // _WPB
    mg = (wid % _WPB) * _BPW

    pltpu.sync_copy(xs.at[b], xs_v)
    pltpu.sync_copy(ys.at[b], ys_v)
    pltpu.sync_copy(zs.at[b], zs_v)

    lane = lax.iota(jnp.int32, _L)
    ones_v = jnp.full((_L,), 1, jnp.int32)
    zeros_v = jnp.full((_L,), 0, jnp.int32)
    s_cap = jnp.full((_L,), _S, jnp.int32)
    d_v = jnp.full((_L,), _D, jnp.int32)
    bn_v = jnp.full((_L,), b * _N, jnp.int32)
    seven_v = jnp.full((_L,), 7, jnp.int32)
    m127_v = jnp.full((_L,), 127, jnp.int32)
    fzero_v = jnp.zeros((_L,), jnp.float32)

    def box_body(k, flag_vec):
        m = mg + k
        pltpu.sync_copy(boxp.at[b, m], boxp_v)
        cx = boxp_v[0]
        cy = boxp_v[1]
        cz = boxp_v[2]
        hx = boxp_v[3]
        hy = boxp_v[4]
        hz = boxp_v[5]
        ca = boxp_v[6]
        sa = boxp_v[7]

        def scan_cond(carry):
            i, off_v = carry
            return (i < _NIT) & jnp.all(off_v < s_cap)

        def scan_body(carry):
            i, off_v = carry
            base = i * (_CHUNK * _L)
            for t in range(_CHUNK):
                xv = xs_v[i, pl.ds(t * _L, _L)]
                yv = ys_v[i, pl.ds(t * _L, _L)]
                zv = zs_v[i, pl.ds(t * _L, _L)]
                px = xv - cx
                py = yv - cy
                pz = zv - cz
                lx = px * ca + py * sa
                ly = py * ca - px * sa
                mask = ((jnp.abs(lx) < hx) & (jnp.abs(ly) < hy)
                        & (jnp.abs(pz) <= hz))
                mi = jnp.where(mask, ones_v, zeros_v)
                pos = plsc.cumsum(mi) + off_v - ones_v
                ok = mask & (pos < s_cap)
                ptid = lane + jnp.full((_L,), t * _L, jnp.int32) \
                    + jnp.full((_L,), base, jnp.int32)
                plsc.store_scatter(idx_buf, [pos], ptid, mask=ok)
                off_v = off_v + plsc.all_reduce_population_count(mask)
            return i + 1, off_v

        _, cnt_all_v = lax.while_loop(
            scan_cond, scan_body, (jnp.int32(0), jnp.zeros((_L,), jnp.int32)))

        cnt_v = jnp.minimum(cnt_all_v, s_cap)
        cntc_v = jnp.maximum(cnt_v, ones_v)
        nonempty = cnt_v > zeros_v
        ne = jnp.all(nonempty)
        for j in range(_S // _L):
            sv = lane + jnp.full((_L,), j * _L, jnp.int32)
            rv = jnp.remainder(sv, cntc_v)
            g = plsc.load_gather(idx_buf, [rv])
            g = jnp.where(nonempty, g + bn_v, bn_v)
            pidx_buf[j // 8, pl.ds((j % 8) * _L, _L)] = g

        def gdesc(q):
            return pltpu.make_async_copy(
                feat.at[pidx_buf.at[q]], rows_f.at[q % 2], gsem)

        def odesc(mm, q):
            return pltpu.make_async_copy(
                flat_v, out.at[b, mm, pl.ds(q * _Q, _Q)], osem)

        pltpu.async_copy(feat.at[pidx_buf.at[0]], rows_f.at[0], gsem)
        for q in range(4):
            qb = q % 2
            gdesc(q).wait()
            if q < 3:
                pltpu.async_copy(feat.at[pidx_buf.at[q + 1]],
                                 rows_f.at[(q + 1) % 2], gsem)
            if q >= 1:
                odesc(m, q - 1).wait()
            else:
                @pl.when(k > 0)
                def _():
                    odesc(m - 1, 3).wait()

            @pl.when(ne)
            def _():
                def frow(s4, rbase):
                    for r in range(4):
                        s = s4 * 4 + r
                        for j2 in range(8):
                            v = rows_f[qb, s, pl.ds(j2 * _L, _L)]
                            cidx = lane + jnp.full((_L,), 3 + j2 * _L,
                                                   jnp.int32)
                            ridx = rbase + jnp.full((_L,), r, jnp.int32)
                            plsc.store_scatter(flat_v, [ridx, cidx], v)
                    return rbase + jnp.full((_L,), 4, jnp.int32)

                lax.fori_loop(0, _Q // 4, frow, jnp.zeros((_L,), jnp.int32))
                for t in range(_Q // _L):
                    pv = pidx_buf[q, pl.ds(t * _L, _L)] - bn_v
                    rr = jax.lax.shift_right_logical(pv, seven_v)
                    cc = pv & m127_v
                    xg = plsc.load_gather(xs_v, [rr, cc])
                    yg = plsc.load_gather(ys_v, [rr, cc])
                    zg = plsc.load_gather(zs_v, [rr, cc])
                    sv = lane + jnp.full((_L,), t * _L, jnp.int32)
                    plsc.store_scatter(flat_v, [sv, zeros_v], xg)
                    plsc.store_scatter(flat_v, [sv, ones_v], yg)
                    plsc.store_scatter(flat_v, [sv, ones_v + ones_v], zg)

            @pl.when(jnp.logical_not(ne))
            def _():
                tail_ok = lane < jnp.full((_L,), 3, jnp.int32)

                def zrow(s, acc):
                    for j2 in range(8):
                        flat_v[s, pl.ds(j2 * _L, _L)] = fzero_v
                    tidx = lane + jnp.full((_L,), 128, jnp.int32)
                    plsc.store_scatter(flat_v,
                                       [jnp.full((_L,), s, jnp.int32), tidx],
                                       fzero_v, mask=tail_ok)
                    return acc

                lax.fori_loop(0, _Q, zrow, jnp.int32(0))

            pltpu.async_copy(flat_v, out.at[b, m, pl.ds(q * _Q, _Q)], osem)

        flag = jnp.where(nonempty, zeros_v, ones_v)
        k_v = jnp.full((_L,), k, jnp.int32)
        return jnp.where(lane == k_v, flag, flag_vec)

    flags = lax.fori_loop(0, _BPW, box_body, jnp.zeros((_L,), jnp.int32))
    m_last = mg + _BPW - 1
    pltpu.make_async_copy(
        flat_v, out.at[b, m_last, pl.ds(3 * _Q, _Q)], osem).wait()
    flags_v[...] = flags
    pltpu.sync_copy(flags_v, flags_out.at[b, pl.ds(mg, _L)])


@functools.partial(
    pl.kernel,
    mesh=plsc.VectorSubcoreMesh(core_axis_name="c", subcore_axis_name="s"),
    compiler_params=pltpu.CompilerParams(needs_layout_passes=False,
                                         use_tc_tiling_on_sc=True),
    out_type=[
        jax.ShapeDtypeStruct((_B, _M, _S, _D), jnp.float32),
        jax.ShapeDtypeStruct((_B, _M), jnp.int32),
    ],
    scratch_types=[
        pltpu.VMEM((_N // 128, 128), jnp.float32),
        pltpu.VMEM((_N // 128, 128), jnp.float32),
        pltpu.VMEM((_N // 128, 128), jnp.float32),
        pltpu.VMEM((8, _L), jnp.float32),
        pltpu.VMEM((_S,), jnp.int32),
        pltpu.VMEM((_S // 128, 128), jnp.int32),
        pltpu.VMEM((2, _Q, 128), jnp.float32),
        pltpu.VMEM((_Q, _D), jnp.float32),
        pltpu.VMEM((_L,), jnp.int32),
        pltpu.SemaphoreType.DMA,
        pltpu.SemaphoreType.DMA,
    ],
)
def _sc_pool(*args):
    _sc_body(*args)


def kernel(points, point_features, boxes3d):
    points = points.astype(jnp.float32)
    point_features = point_features.astype(jnp.float32)
    boxes3d = boxes3d.astype(jnp.float32)

    xs = points[..., 0].reshape(_B, _N // 128, 128)
    ys = points[..., 1].reshape(_B, _N // 128, 128)
    zs = points[..., 2].reshape(_B, _N // 128, 128)
    half = boxes3d[..., 3:6] * 0.5 + _EXTRA
    rz = boxes3d[..., 6]
    boxp = jnp.stack(
        [boxes3d[..., 0], boxes3d[..., 1], boxes3d[..., 2],
         half[..., 0], half[..., 1], half[..., 2],
         jnp.cos(rz), jnp.sin(rz)], axis=-1)
    boxp = (boxp[..., None] * jnp.ones((_L,), jnp.float32))

    feat = point_features.reshape(_B * _N, _C)

    pooled, flags = _sc_pool(xs, ys, zs, boxp, feat)
    return pooled, flags
